# Initial kernel scaffold; baseline (speedup 1.0000x reference)
import jax
import jax.numpy as jnp
from jax.experimental import pallas as pl
from jax.experimental.pallas import tpu as pltpu

_R = 512
_DBLK = 512


def _body(x_ref, o_ref, carry_ref):
    i = pl.program_id(1)

    @pl.when(i == 0)
    def _():
        carry_ref[...] = jnp.zeros_like(carry_ref)

    x = x_ref[0]

    rows = jax.lax.broadcasted_iota(jnp.int32, (_R, _R), 0)
    cols = jax.lax.broadcasted_iota(jnp.int32, (_R, _R), 1)
    tri = (rows >= cols).astype(jnp.float32)

    acc = jnp.dot(tri, x, preferred_element_type=jnp.float32) + carry_ref[...]

    local = jax.lax.broadcasted_iota(jnp.float32, (_R, 1), 0)
    denom = local + (i * _R + 1).astype(jnp.float32)
    o_ref[0] = acc * (1.0 / denom)

    carry_ref[...] = carry_ref[...] + jnp.sum(x, axis=0, keepdims=True)


@jax.jit
def kernel(x):
    b, s, d = x.shape
    nd = d // _DBLK
    grid = (b * nd, s // _R)
    return pl.pallas_call(
        _body,
        grid=grid,
        in_specs=[
            pl.BlockSpec((1, _R, _DBLK), lambda p, i: (p // nd, i, p % nd))
        ],
        out_specs=pl.BlockSpec((1, _R, _DBLK), lambda p, i: (p // nd, i, p % nd)),
        out_shape=jax.ShapeDtypeStruct((b, s, d), x.dtype),
        scratch_shapes=[pltpu.VMEM((1, _DBLK), jnp.float32)],
        compiler_params=pltpu.CompilerParams(
            dimension_semantics=("parallel", "arbitrary"),
        ),
    )(x)


# speedup vs baseline: 2.3739x; 2.3739x over previous
import jax
import jax.numpy as jnp
from jax.experimental import pallas as pl
from jax.experimental.pallas import tpu as pltpu

_R = 512
_DBLK = 512


def _body(x_ref, o_ref, carry_ref):
    i = pl.program_id(1)

    @pl.when(i == 0)
    def _():
        carry_ref[...] = jnp.zeros_like(carry_ref)

    x = x_ref[0]

    rows = jax.lax.broadcasted_iota(jnp.int32, (_R, _R), 0)
    cols = jax.lax.broadcasted_iota(jnp.int32, (_R, _R), 1)
    tri = (rows >= cols).astype(jnp.float32)

    acc = jnp.dot(tri, x, preferred_element_type=jnp.float32) + carry_ref[...]

    local = jax.lax.broadcasted_iota(jnp.int32, (_R, 1), 0)
    denom = (local + (i * _R + 1)).astype(jnp.float32)
    o_ref[0] = acc * (1.0 / denom)

    carry_ref[...] = carry_ref[...] + jnp.sum(x, axis=0, keepdims=True)


@jax.jit
def kernel(x):
    b, s, d = x.shape
    nd = d // _DBLK
    grid = (b * nd, s // _R)
    return pl.pallas_call(
        _body,
        grid=grid,
        in_specs=[
            pl.BlockSpec((1, _R, _DBLK), lambda p, i: (p // nd, i, p % nd))
        ],
        out_specs=pl.BlockSpec((1, _R, _DBLK), lambda p, i: (p // nd, i, p % nd)),
        out_shape=jax.ShapeDtypeStruct((b, s, d), x.dtype),
        scratch_shapes=[pltpu.VMEM((1, _DBLK), jnp.float32)],
        compiler_params=pltpu.CompilerParams(
            dimension_semantics=("parallel", "arbitrary"),
        ),
    )(x)


# speedup vs baseline: 2.5141x; 1.0591x over previous
import jax
import jax.numpy as jnp
from jax.experimental import pallas as pl
from jax.experimental.pallas import tpu as pltpu

_R = 512
_DBLK = 512


_T = 256


def _body(x_ref, o_ref, carry_ref):
    i = pl.program_id(1)

    @pl.when(i == 0)
    def _():
        carry_ref[...] = jnp.zeros_like(carry_ref)

    rows = jax.lax.broadcasted_iota(jnp.int32, (_T, _T), 0)
    cols = jax.lax.broadcasted_iota(jnp.int32, (_T, _T), 1)
    tri = jnp.where(rows.astype(jnp.bfloat16) >= cols.astype(jnp.bfloat16),
                    jnp.bfloat16(1), jnp.bfloat16(0))

    xb = x_ref[0].astype(jnp.bfloat16)

    carry = carry_ref[...]
    for j in range(_R // _T):
        sub = xb[j * _T:(j + 1) * _T]
        p = jnp.dot(tri, sub, preferred_element_type=jnp.float32)
        local = jax.lax.broadcasted_iota(jnp.int32, (_T, 1), 0)
        denom = (local + (i * _R + j * _T + 1)).astype(jnp.float32)
        o_ref[0, j * _T:(j + 1) * _T, :] = (p + carry) * (1.0 / denom)
        carry = carry + p[_T - 1:_T, :]
    carry_ref[...] = carry


@jax.jit
def kernel(x):
    b, s, d = x.shape
    nd = d // _DBLK
    grid = (b * nd, s // _R)
    return pl.pallas_call(
        _body,
        grid=grid,
        in_specs=[
            pl.BlockSpec((1, _R, _DBLK), lambda p, i: (p // nd, i, p % nd))
        ],
        out_specs=pl.BlockSpec((1, _R, _DBLK), lambda p, i: (p // nd, i, p % nd)),
        out_shape=jax.ShapeDtypeStruct((b, s, d), x.dtype),
        scratch_shapes=[pltpu.VMEM((1, _DBLK), jnp.float32)],
        compiler_params=pltpu.CompilerParams(
            dimension_semantics=("parallel", "arbitrary"),
        ),
    )(x)


# speedup vs baseline: 4.2497x; 1.6903x over previous
import jax
import jax.numpy as jnp
from jax.experimental import pallas as pl
from jax.experimental.pallas import tpu as pltpu

_R = 4096
_DBLK = 512


_T = 256


def _body(x_ref, o_ref, carry_ref):
    i = pl.program_id(1)

    @pl.when(i == 0)
    def _():
        carry_ref[...] = jnp.zeros_like(carry_ref)

    rows = jax.lax.broadcasted_iota(jnp.int32, (_T, _T), 0)
    cols = jax.lax.broadcasted_iota(jnp.int32, (_T, _T), 1)
    tri = jnp.where(rows.astype(jnp.bfloat16) >= cols.astype(jnp.bfloat16),
                    jnp.bfloat16(1), jnp.bfloat16(0))

    xb = x_ref[0].astype(jnp.bfloat16)

    carry = carry_ref[...]
    for j in range(_R // _T):
        sub = xb[j * _T:(j + 1) * _T]
        p = jnp.dot(tri, sub, preferred_element_type=jnp.float32)
        local = jax.lax.broadcasted_iota(jnp.int32, (_T, 1), 0)
        denom = (local + (i * _R + j * _T + 1)).astype(jnp.float32)
        o_ref[0, j * _T:(j + 1) * _T, :] = (p + carry) * (1.0 / denom)
        carry = carry + p[_T - 1:_T, :]
    carry_ref[...] = carry


@jax.jit
def kernel(x):
    b, s, d = x.shape
    nd = d // _DBLK
    grid = (b * nd, s // _R)
    return pl.pallas_call(
        _body,
        grid=grid,
        in_specs=[
            pl.BlockSpec((1, _R, _DBLK), lambda p, i: (p // nd, i, p % nd))
        ],
        out_specs=pl.BlockSpec((1, _R, _DBLK), lambda p, i: (p // nd, i, p % nd)),
        out_shape=jax.ShapeDtypeStruct((b, s, d), x.dtype),
        scratch_shapes=[pltpu.VMEM((1, _DBLK), jnp.float32)],
        compiler_params=pltpu.CompilerParams(
            dimension_semantics=("parallel", "arbitrary"),
            vmem_limit_bytes=56 * 1024 * 1024,
        ),
    )(x)
